# Initial kernel scaffold; baseline (speedup 1.0000x reference)
import jax
import jax.numpy as jnp
from jax.experimental import pallas as pl


def kernel(x, se_w1, se_w2, convh_w, convh_b, dyh_fc1_w, dyh_fc1_b, dyh_fc2_w, dyh_fc2_b, convw_w, convw_b, dyw_fc1_w, dyw_fc1_b, dyw_fc2_w, dyw_fc2_b):
    raise NotImplementedError("write your pallas kernel here")


# speedup vs baseline: 320.6258x; 320.6258x over previous
import jax
import jax.numpy as jnp
from jax import lax
from jax.experimental import pallas as pl
from jax.experimental.pallas import tpu as pltpu

_B, _C, _H, _W = 64, 256, 64, 32
_HW = _H * _W
_RH, _RW = 3, 3


def _sigmoid(v):
    return 1.0 / (1.0 + jnp.exp(-v))


def _head(g, conv_w, conv_b, fc1_w, fc1_b, fc2s_w, fc2s_b, limit):
    s = jnp.dot(conv_w, g, preferred_element_type=jnp.float32) + conv_b
    th = jax.nn.relu(jnp.dot(fc1_w, s, preferred_element_type=jnp.float32) + fc1_b)
    t0 = jnp.dot(fc2s_w[0], th, preferred_element_type=jnp.float32) + fc2s_b[0]
    t1 = jnp.dot(fc2s_w[1], th, preferred_element_type=jnp.float32) + fc2s_b[1]
    t2 = jnp.dot(fc2s_w[2], th, preferred_element_type=jnp.float32) + fc2s_b[2]
    t3 = jnp.dot(fc2s_w[3], th, preferred_element_type=jnp.float32) + fc2s_b[3]
    a1 = 2.0 * _sigmoid(t0)
    a2 = 2.0 * _sigmoid(t1) - 1.0
    b1 = _sigmoid(t2) - 0.5
    b2 = _sigmoid(t3) - 0.5
    dy = jnp.maximum(s * a1 + b1, s * a2 + b2)
    coord = jnp.ceil(_H * _sigmoid(dy))
    coord = jnp.minimum(coord, float(limit))
    return coord.astype(jnp.int32)


def _body(x_ref, se_w1_ref, se_w2_ref,
          convh_w_ref, convh_b_ref, fc1h_w_ref, fc1h_b_ref, fc2h_w_ref, fc2h_b_ref,
          convw_w_ref, convw_b_ref, fc1w_w_ref, fc1w_b_ref, fc2w_w_ref, fc2w_b_ref,
          out_ref):
    x = x_ref[...]
    m = jnp.sum(x, axis=1, keepdims=True) * (1.0 / _HW)
    t = jax.nn.relu(jnp.dot(se_w1_ref[...], m, preferred_element_type=jnp.float32))
    y = _sigmoid(jnp.dot(se_w2_ref[...], t, preferred_element_type=jnp.float32))
    g = m * y
    sx = _head(g, convh_w_ref[...], convh_b_ref[...], fc1h_w_ref[...], fc1h_b_ref[...],
               fc2h_w_ref, fc2h_b_ref[...], _H - _RH)
    sy = _head(g, convw_w_ref[...], convw_b_ref[...], fc1w_w_ref[...], fc1w_b_ref[...],
               fc2w_w_ref, fc2w_b_ref[...], _W - _RW)
    hw = lax.broadcasted_iota(jnp.int32, (_C, _HW), 1)
    i = hw >> 5
    j = hw & (_W - 1)
    row_bad = (i - sx).astype(jnp.uint32) < jnp.uint32(_RH)
    col_bad = (j - sy).astype(jnp.uint32) < jnp.uint32(_RW)
    out_ref[...] = jnp.where(row_bad & col_bad, 0.0, x * y)


def kernel(x, se_w1, se_w2, convh_w, convh_b, dyh_fc1_w, dyh_fc1_b, dyh_fc2_w, dyh_fc2_b,
           convw_w, convw_b, dyw_fc1_w, dyw_fc1_b, dyw_fc2_w, dyw_fc2_b):
    b, c, h, w = x.shape
    x2 = x.reshape(b, c, h * w)
    col = lambda v: v.reshape(-1, 1)
    fc2h = dyh_fc2_w.reshape(c, 4, -1).transpose(1, 0, 2)
    fc2hb = dyh_fc2_b.reshape(c, 4).T.reshape(4, c, 1)
    fc2w = dyw_fc2_w.reshape(c, 4, -1).transpose(1, 0, 2)
    fc2wb = dyw_fc2_b.reshape(c, 4).T.reshape(4, c, 1)

    full = lambda a: pl.BlockSpec(a.shape, lambda i: (0,) * a.ndim)
    operands = (se_w1, se_w2, convh_w, col(convh_b), dyh_fc1_w, col(dyh_fc1_b),
                fc2h, fc2hb, convw_w, col(convw_b), dyw_fc1_w, col(dyw_fc1_b),
                fc2w, fc2wb)
    out = pl.pallas_call(
        _body,
        grid=(b,),
        in_specs=[pl.BlockSpec((None, c, h * w), lambda i: (i, 0, 0))]
        + [full(a) for a in operands],
        out_specs=pl.BlockSpec((None, c, h * w), lambda i: (i, 0, 0)),
        out_shape=jax.ShapeDtypeStruct((b, c, h * w), jnp.float32),
        compiler_params=pltpu.CompilerParams(
            dimension_semantics=("arbitrary",),
        ),
    )(x2, *operands)
    return out.reshape(b, c, h, w)


# speedup vs baseline: 1158.3792x; 3.6129x over previous
import jax
import jax.numpy as jnp
from jax import lax
from jax.experimental import pallas as pl
from jax.experimental.pallas import tpu as pltpu

_B, _C, _H, _W = 64, 256, 64, 32
_RH, _RW = 3, 3
_NB = 2


def _sigmoid(v):
    return 1.0 / (1.0 + jnp.exp(-v))


def _body(x_ref, w1t_ref, w2t_ref, wcv_ref, bcv_ref, wf1_ref, bf1_ref,
          wf2_ref, bf2_ref, out_ref):
    x = x_ref[...]
    m = jnp.sum(jnp.sum(x, axis=1), axis=1) * (1.0 / (_H * _W))
    t = jax.nn.relu(jnp.dot(m, w1t_ref[...], preferred_element_type=jnp.float32))
    y = _sigmoid(jnp.dot(t, w2t_ref[...], preferred_element_type=jnp.float32))
    g = m * y
    s2 = jnp.dot(g, wcv_ref[...], preferred_element_type=jnp.float32) + bcv_ref[...]
    th = jax.nn.relu(jnp.dot(s2, wf1_ref[...], preferred_element_type=jnp.float32)
                     + bf1_ref[...])
    t2 = jnp.dot(th, wf2_ref[...], preferred_element_type=jnp.float32) + bf2_ref[...]

    def head(s, base, limit):
        a1 = 2.0 * _sigmoid(t2[:, base:base + _C])
        a2 = 2.0 * _sigmoid(t2[:, base + _C:base + 2 * _C]) - 1.0
        b1 = _sigmoid(t2[:, base + 2 * _C:base + 3 * _C]) - 0.5
        b2 = _sigmoid(t2[:, base + 3 * _C:base + 4 * _C]) - 0.5
        dy = jnp.maximum(s * a1 + b1, s * a2 + b2)
        coord = jnp.minimum(jnp.ceil(_H * _sigmoid(dy)), float(limit))
        return coord.astype(jnp.int32)

    sx = head(s2[:, :_C], 0, _H - _RH)
    sy = head(s2[:, _C:], 4 * _C, _W - _RW)

    ii = lax.broadcasted_iota(jnp.int32, (_NB, _H, _W, _C), 1)
    jj = lax.broadcasted_iota(jnp.int32, (_NB, _H, _W, _C), 2)
    row_bad = (ii - sx[:, None, None, :]).astype(jnp.uint32) < jnp.uint32(_RH)
    col_bad = (jj - sy[:, None, None, :]).astype(jnp.uint32) < jnp.uint32(_RW)
    out_ref[...] = jnp.where(row_bad & col_bad, 0.0, x * y[:, None, None, :])


def kernel(x, se_w1, se_w2, convh_w, convh_b, dyh_fc1_w, dyh_fc1_b, dyh_fc2_w, dyh_fc2_b,
           convw_w, convw_b, dyw_fc1_w, dyw_fc1_b, dyw_fc2_w, dyw_fc2_b):
    b, c, h, w = x.shape
    xt = jnp.transpose(x, (0, 2, 3, 1))
    red = dyh_fc1_w.shape[0]
    z = jnp.zeros((c, red), dtype=jnp.float32)
    z2 = jnp.zeros((red, 4 * c), dtype=jnp.float32)
    gm = lambda wf: wf.reshape(c, 4, red).transpose(1, 0, 2).reshape(4 * c, red).T
    wgts = (
        se_w1.T, se_w2.T,
        jnp.concatenate([convh_w.T, convw_w.T], axis=1),
        jnp.concatenate([convh_b, convw_b]).reshape(1, 2 * c),
        jnp.concatenate([jnp.concatenate([dyh_fc1_w.T, z], axis=1),
                         jnp.concatenate([z, dyw_fc1_w.T], axis=1)], axis=0),
        jnp.concatenate([dyh_fc1_b, dyw_fc1_b]).reshape(1, 2 * red),
        jnp.concatenate([jnp.concatenate([gm(dyh_fc2_w), z2], axis=1),
                         jnp.concatenate([z2, gm(dyw_fc2_w)], axis=1)], axis=0),
        jnp.concatenate([dyh_fc2_b.reshape(c, 4).T.reshape(-1),
                         dyw_fc2_b.reshape(c, 4).T.reshape(-1)]).reshape(1, 8 * c),
    )
    full = lambda a: pl.BlockSpec(a.shape, lambda i: (0,) * a.ndim)
    outt = pl.pallas_call(
        _body,
        grid=(b // _NB,),
        in_specs=[pl.BlockSpec((_NB, h, w, c), lambda i: (i, 0, 0, 0))]
        + [full(a) for a in wgts],
        out_specs=pl.BlockSpec((_NB, h, w, c), lambda i: (i, 0, 0, 0)),
        out_shape=jax.ShapeDtypeStruct((b, h, w, c), jnp.float32),
        compiler_params=pltpu.CompilerParams(
            dimension_semantics=("arbitrary",),
        ),
    )(xt, *wgts)
    return jnp.transpose(outt, (0, 3, 1, 2))


# speedup vs baseline: 1204.4969x; 1.0398x over previous
import jax
import jax.numpy as jnp
from jax import lax
from jax.experimental import pallas as pl
from jax.experimental.pallas import tpu as pltpu

_B, _C, _H, _W = 64, 256, 64, 32
_RH, _RW = 3, 3
_NB = 2


def _sigmoid(v):
    return 1.0 / (1.0 + jnp.exp(-v))


def _dgt(a, b):
    return lax.dot_general(a, b, (((1,), (1,)), ((), ())),
                           preferred_element_type=jnp.float32)


def _head(g, conv_w, conv_b, fc1_w, fc1_b, fc2g_ref, fc2b, limit):
    s = _dgt(g, conv_w) + conv_b[None, :]
    th = jax.nn.relu(_dgt(s, fc1_w) + fc1_b[None, :])
    a1 = 2.0 * _sigmoid(_dgt(th, fc2g_ref[0]) + fc2b[0][None, :])
    a2 = 2.0 * _sigmoid(_dgt(th, fc2g_ref[1]) + fc2b[1][None, :]) - 1.0
    b1 = _sigmoid(_dgt(th, fc2g_ref[2]) + fc2b[2][None, :]) - 0.5
    b2 = _sigmoid(_dgt(th, fc2g_ref[3]) + fc2b[3][None, :]) - 0.5
    dy = jnp.maximum(s * a1 + b1, s * a2 + b2)
    coord = jnp.minimum(jnp.ceil(_H * _sigmoid(dy)), float(limit))
    return coord.astype(jnp.int32)


def _body(x_ref, w1_ref, w2_ref, convh_w_ref, convh_b_ref, fc1h_w_ref, fc1h_b_ref,
          fc2gh_ref, fc2bh_ref, convw_w_ref, convw_b_ref, fc1w_w_ref, fc1w_b_ref,
          fc2gw_ref, fc2bw_ref, out_ref):
    x = x_ref[...]
    m = jnp.sum(jnp.sum(x, axis=1), axis=1) * (1.0 / (_H * _W))
    t = jax.nn.relu(_dgt(m, w1_ref[...]))
    y = _sigmoid(_dgt(t, w2_ref[...]))
    g = m * y
    sx = _head(g, convh_w_ref[...], convh_b_ref[...], fc1h_w_ref[...],
               fc1h_b_ref[...], fc2gh_ref, fc2bh_ref[...], _H - _RH)
    sy = _head(g, convw_w_ref[...], convw_b_ref[...], fc1w_w_ref[...],
               fc1w_b_ref[...], fc2gw_ref, fc2bw_ref[...], _W - _RW)

    ii = lax.broadcasted_iota(jnp.int32, (_NB, _H, _W, _C), 1)
    jj = lax.broadcasted_iota(jnp.int32, (_NB, _H, _W, _C), 2)
    row_bad = (ii - sx[:, None, None, :]).astype(jnp.uint32) < jnp.uint32(_RH)
    col_bad = (jj - sy[:, None, None, :]).astype(jnp.uint32) < jnp.uint32(_RW)
    out_ref[...] = jnp.where(row_bad & col_bad, 0.0, x * y[:, None, None, :])


def kernel(x, se_w1, se_w2, convh_w, convh_b, dyh_fc1_w, dyh_fc1_b, dyh_fc2_w, dyh_fc2_b,
           convw_w, convw_b, dyw_fc1_w, dyw_fc1_b, dyw_fc2_w, dyw_fc2_b):
    b, c, h, w = x.shape
    xt = jnp.transpose(x, (0, 2, 3, 1))
    red = dyh_fc1_w.shape[0]
    gm = lambda wf: wf.reshape(c, 4, red).transpose(1, 0, 2)
    gb = lambda bf: bf.reshape(c, 4).T
    wgts = (se_w1, se_w2,
            convh_w, convh_b, dyh_fc1_w, dyh_fc1_b, gm(dyh_fc2_w), gb(dyh_fc2_b),
            convw_w, convw_b, dyw_fc1_w, dyw_fc1_b, gm(dyw_fc2_w), gb(dyw_fc2_b))
    full = lambda a: pl.BlockSpec(a.shape, lambda i: (0,) * a.ndim)
    outt = pl.pallas_call(
        _body,
        grid=(b // _NB,),
        in_specs=[pl.BlockSpec((_NB, h, w, c), lambda i: (i, 0, 0, 0))]
        + [full(a) for a in wgts],
        out_specs=pl.BlockSpec((_NB, h, w, c), lambda i: (i, 0, 0, 0)),
        out_shape=jax.ShapeDtypeStruct((b, h, w, c), jnp.float32),
        compiler_params=pltpu.CompilerParams(
            dimension_semantics=("arbitrary",),
        ),
    )(xt, *wgts)
    return jnp.transpose(outt, (0, 3, 1, 2))
